# baseline (speedup 1.0000x reference)
import jax
import jax.numpy as jnp
from jax.experimental import pallas as pl
from jax.experimental.pallas import tpu as pltpu


def _copy_block(src_ref, dst_ref):
    dst_ref[...] = src_ref[...]


def _pallas_copy(x, block_rows):
    rows, cols = x.shape
    grid = (rows // block_rows,)
    return pl.pallas_call(
        _copy_block,
        grid=grid,
        in_specs=[pl.BlockSpec((block_rows, cols), lambda i: (i, 0))],
        out_specs=pl.BlockSpec((block_rows, cols), lambda i: (i, 0)),
        out_shape=jax.ShapeDtypeStruct((rows, cols), x.dtype),
    )(x)


def kernel(user_w, item_w, edge_index):
    del edge_index
    u2 = user_w.reshape(50000, 128)
    i2 = item_w.reshape(500000, 128)
    user_out = _pallas_copy(u2, block_rows=10000)
    item_out = _pallas_copy(i2, block_rows=10000)
    return (user_out.reshape(user_w.shape), item_out.reshape(item_w.shape))


# speedup vs baseline: 1.2423x; 1.2423x over previous
import functools

import jax
import jax.numpy as jnp
from jax import lax
from jax.experimental import pallas as pl
from jax.experimental.pallas import tpu as pltpu
from jax.experimental.pallas import tpu_sc as plsc

_RC = 400
_NI = 2500
_NT = 2750
_NW = 32
_SLOTS = 86


def _sc_copy(user_w, item_w):
    mesh = plsc.VectorSubcoreMesh(core_axis_name="c", subcore_axis_name="s")

    @functools.partial(
        pl.kernel,
        mesh=mesh,
        out_type=[
            jax.ShapeDtypeStruct(user_w.shape, user_w.dtype),
            jax.ShapeDtypeStruct(item_w.shape, item_w.dtype),
        ],
        scratch_types=(
            [pltpu.VMEM((_RC, 64), jnp.float32)] * 2
            + [pltpu.SemaphoreType.DMA] * 4
        ),
    )
    def sc_kernel(u_hbm, i_hbm, uo_hbm, io_hbm,
                  buf0, buf1, isem0, isem1, osem0, osem1):
        bufs = (buf0, buf1)
        isems = (isem0, isem1)
        osems = (osem0, osem1)
        wid = lax.axis_index("c") * 16 + lax.axis_index("s")

        def start_chunk(k, b):
            @pl.when(k < _NI)
            def _():
                sl = pl.ds(k * _RC, _RC)
                pltpu.make_async_copy(i_hbm.at[sl], bufs[b], isems[b]).start()

            @pl.when((k >= _NI) & (k < _NT))
            def _():
                sl = pl.ds((k - _NI) * _RC, _RC)
                pltpu.make_async_copy(u_hbm.at[sl], bufs[b], isems[b]).start()

        def finish_chunk(k, b):
            @pl.when(k < _NI)
            def _():
                sl = pl.ds(k * _RC, _RC)
                pltpu.make_async_copy(i_hbm.at[sl], bufs[b], isems[b]).wait()
                pltpu.make_async_copy(bufs[b], io_hbm.at[sl], osems[b]).start()

            @pl.when((k >= _NI) & (k < _NT))
            def _():
                sl = pl.ds((k - _NI) * _RC, _RC)
                pltpu.make_async_copy(u_hbm.at[sl], bufs[b], isems[b]).wait()
                pltpu.make_async_copy(bufs[b], uo_hbm.at[sl], osems[b]).start()

        def wait_out(k, b):
            @pl.when((k >= 0) & (k < _NT))
            def _():
                pltpu.make_async_copy(
                    bufs[b], io_hbm.at[pl.ds(0, _RC)], osems[b]).wait()

        def body(j, carry):
            for b in range(2):
                k = (2 * j + b) * _NW + wid
                wait_out(k - 2 * _NW, b)
                start_chunk(k, b)
            for b in range(2):
                k = (2 * j + b) * _NW + wid
                finish_chunk(k, b)
            return carry

        lax.fori_loop(0, _SLOTS // 2, body, 0)
        for b in range(2):
            wait_out((_SLOTS - 2 + b) * _NW + wid, b)

    return sc_kernel(user_w, item_w)


def kernel(user_w, item_w, edge_index):
    del edge_index
    user_out, item_out = _sc_copy(user_w, item_w)
    return (user_out, item_out)


# speedup vs baseline: 1.2477x; 1.0043x over previous
import functools

import jax
import jax.numpy as jnp
from jax import lax
from jax.experimental import pallas as pl
from jax.experimental.pallas import tpu as pltpu
from jax.experimental.pallas import tpu_sc as plsc

_RC = 200
_NI = 5000
_NT = 5500
_NW = 32
_NB = 4
_S = 172


def _sc_copy(user_w, item_w):
    mesh = plsc.VectorSubcoreMesh(core_axis_name="c", subcore_axis_name="s")

    @functools.partial(
        pl.kernel,
        mesh=mesh,
        out_type=[
            jax.ShapeDtypeStruct(user_w.shape, user_w.dtype),
            jax.ShapeDtypeStruct(item_w.shape, item_w.dtype),
        ],
        scratch_types=(
            [pltpu.VMEM((_RC, 64), jnp.float32)] * _NB
            + [pltpu.SemaphoreType.DMA] * (2 * _NB)
        ),
    )
    def sc_kernel(u_hbm, i_hbm, uo_hbm, io_hbm, *scr):
        bufs = scr[:_NB]
        isems = scr[_NB:2 * _NB]
        osems = scr[2 * _NB:3 * _NB]
        wid = lax.axis_index("c") * 16 + lax.axis_index("s")

        def valid(s):
            return (s >= 0) & (s * _NW + wid < _NT)

        def start_in(s, b):
            k = s * _NW + wid

            @pl.when(valid(s) & (k < _NI))
            def _():
                sl = pl.ds(k * _RC, _RC)
                pltpu.make_async_copy(i_hbm.at[sl], bufs[b], isems[b]).start()

            @pl.when(valid(s) & (k >= _NI))
            def _():
                sl = pl.ds((k - _NI) * _RC, _RC)
                pltpu.make_async_copy(u_hbm.at[sl], bufs[b], isems[b]).start()

        def finish_start_out(s, b):
            k = s * _NW + wid

            @pl.when(valid(s) & (k < _NI))
            def _():
                sl = pl.ds(k * _RC, _RC)
                pltpu.make_async_copy(i_hbm.at[sl], bufs[b], isems[b]).wait()
                pltpu.make_async_copy(bufs[b], io_hbm.at[sl], osems[b]).start()

            @pl.when(valid(s) & (k >= _NI))
            def _():
                sl = pl.ds((k - _NI) * _RC, _RC)
                pltpu.make_async_copy(u_hbm.at[sl], bufs[b], isems[b]).wait()
                pltpu.make_async_copy(bufs[b], uo_hbm.at[sl], osems[b]).start()

        def wait_out(s, b):
            @pl.when(valid(s))
            def _():
                pltpu.make_async_copy(
                    bufs[b], io_hbm.at[pl.ds(0, _RC)], osems[b]).wait()

        start_in(0, 0)
        start_in(1, 1)

        def body(j, carry):
            for b in range(_NB):
                s = _NB * j + b
                finish_start_out(s, b)
                ns = s + 2
                nb = (b + 2) % _NB
                wait_out(ns - _NB, nb)
                start_in(ns, nb)
            return carry

        lax.fori_loop(0, _S // _NB, body, 0)
        for s in (_S - 2, _S - 1):
            wait_out(s, s % _NB)

    return sc_kernel(user_w, item_w)


def kernel(user_w, item_w, edge_index):
    del edge_index
    user_out, item_out = _sc_copy(user_w, item_w)
    return (user_out, item_out)


# speedup vs baseline: 1.3032x; 1.0445x over previous
import jax
import jax.numpy as jnp
from jax.experimental import pallas as pl
from jax.experimental.pallas import tpu as pltpu


def _copy_block(src_ref, dst_ref):
    dst_ref[...] = src_ref[...]


def _pallas_copy(x, block_rows):
    rows, cols = x.shape
    grid = (rows // block_rows,)
    return pl.pallas_call(
        _copy_block,
        grid=grid,
        in_specs=[pl.BlockSpec((block_rows, cols), lambda i: (i, 0))],
        out_specs=pl.BlockSpec((block_rows, cols), lambda i: (i, 0)),
        out_shape=jax.ShapeDtypeStruct((rows, cols), x.dtype),
        compiler_params=pltpu.CompilerParams(
            dimension_semantics=("parallel",)),
    )(x)


def kernel(user_w, item_w, edge_index):
    del edge_index
    user_out = _pallas_copy(user_w, block_rows=5000)
    item_out = _pallas_copy(item_w, block_rows=20000)
    return (user_out, item_out)
